# Initial kernel scaffold; baseline (speedup 1.0000x reference)
import functools

import jax
import jax.numpy as jnp
from jax import lax
from jax.experimental import pallas as pl
from jax.experimental.pallas import tpu as pltpu
from jax.experimental.pallas import tpu_sc as plsc

NC = 2
NS = 16
NW = NC * NS
CH = 128


def _mesh():
    return plsc.VectorSubcoreMesh(
        core_axis_name="c", subcore_axis_name="s", num_cores=NC, num_subcores=NS
    )


def _make_deg_kernel(n_pad, chunks):
    cpw = chunks // NW
    rpt = n_pad // NS

    def body(dsts, e0, zeros, out, idx_v, e0_v, acc):
        c = lax.axis_index("c")
        s = lax.axis_index("s")
        wid = s * NC + c
        pltpu.sync_copy(zeros.at[pl.ds(s * rpt, rpt)], acc.at[pl.ds(s * rpt, rpt)])
        pltpu.sync_copy(e0, e0_v)
        plsc.subcore_barrier()

        def step(j, carry):
            ch = wid * cpw + j
            pltpu.sync_copy(dsts.at[ch], idx_v)
            pltpu.sync_copy(e0_v, acc.at[idx_v], add=True)
            return carry

        lax.fori_loop(0, cpw, step, 0)
        plsc.subcore_barrier()
        pltpu.sync_copy(acc.at[pl.ds(s * rpt, rpt)], out.at[c, pl.ds(s * rpt, rpt)])

    return pl.kernel(
        body,
        out_type=jax.ShapeDtypeStruct((NC, n_pad, 16), jnp.float32),
        mesh=_mesh(),
        scratch_types=[
            pltpu.VMEM((CH,), jnp.int32),
            pltpu.VMEM((CH, 16), jnp.float32),
            pltpu.VMEM_SHARED((n_pad, 16), jnp.float32),
        ],
    )


def _make_agg_kernel(n_pad, chunks, d):
    cpw = chunks // NW
    rpt = n_pad // NS

    def body(srcs, dsts, y, zeros, out, sidx, didx, rows, acc):
        c = lax.axis_index("c")
        s = lax.axis_index("s")
        wid = s * NC + c
        pltpu.sync_copy(zeros.at[pl.ds(s * rpt, rpt)], acc.at[pl.ds(s * rpt, rpt)])
        plsc.subcore_barrier()

        def step(j, carry):
            ch = wid * cpw + j
            pltpu.sync_copy(srcs.at[ch], sidx)
            pltpu.sync_copy(dsts.at[ch], didx)
            pltpu.sync_copy(y.at[sidx], rows)
            pltpu.sync_copy(rows, acc.at[didx], add=True)
            return carry

        lax.fori_loop(0, cpw, step, 0)
        plsc.subcore_barrier()
        pltpu.sync_copy(acc.at[pl.ds(s * rpt, rpt)], out.at[c, pl.ds(s * rpt, rpt)])

    return pl.kernel(
        body,
        out_type=jax.ShapeDtypeStruct((NC, n_pad, d), jnp.float32),
        mesh=_mesh(),
        scratch_types=[
            pltpu.VMEM((CH,), jnp.int32),
            pltpu.VMEM((CH,), jnp.int32),
            pltpu.VMEM((CH, d), jnp.float32),
            pltpu.VMEM_SHARED((n_pad, d), jnp.float32),
        ],
    )


_DOT = dict(preferred_element_type=jnp.float32, precision=lax.Precision.HIGHEST)


def _tc_pre(degout, x, w1, r):
    n, f = x.shape

    def body(deg_ref, x_ref, w_ref, y_ref, dis_ref):
        cnt = deg_ref[0, :, 0:1] + deg_ref[1, :, 0:1]
        dis = lax.rsqrt(cnt + 1.0)
        y_ref[...] = dis * jnp.dot(x_ref[...], w_ref[...], **_DOT)
        dis_ref[...] = dis

    return pl.pallas_call(
        body,
        grid=(n // r,),
        in_specs=[
            pl.BlockSpec((NC, r, 16), lambda i: (0, i, 0)),
            pl.BlockSpec((r, f), lambda i: (i, 0)),
            pl.BlockSpec((f, f), lambda i: (0, 0)),
        ],
        out_specs=[
            pl.BlockSpec((r, f), lambda i: (i, 0)),
            pl.BlockSpec((r, 1), lambda i: (i, 0)),
        ],
        out_shape=[
            jax.ShapeDtypeStruct((n, f), jnp.float32),
            jax.ShapeDtypeStruct((n, 1), jnp.float32),
        ],
    )(degout, x, w1)


def _tc_mid(agg1, y1, dis, b1, w2, r):
    n, f = y1.shape
    f2 = w2.shape[1]

    def body(a_ref, y1_ref, dis_ref, b_ref, w_ref, y2_ref):
        a = a_ref[0] + a_ref[1]
        dis = dis_ref[...]
        h = jnp.maximum(dis * (a + y1_ref[...]) + b_ref[...], 0.0)
        y2_ref[...] = dis * jnp.dot(h, w_ref[...], **_DOT)

    return pl.pallas_call(
        body,
        grid=(n // r,),
        in_specs=[
            pl.BlockSpec((NC, r, f), lambda i: (0, i, 0)),
            pl.BlockSpec((r, f), lambda i: (i, 0)),
            pl.BlockSpec((r, 1), lambda i: (i, 0)),
            pl.BlockSpec((1, f), lambda i: (0, 0)),
            pl.BlockSpec((f, f2), lambda i: (0, 0)),
        ],
        out_specs=pl.BlockSpec((r, f2), lambda i: (i, 0)),
        out_shape=jax.ShapeDtypeStruct((n, f2), jnp.float32),
    )(agg1, y1, dis, b1, w2)


def _tc_post(agg2, y2, dis, b2, fcw_pad, fcb_pad, r):
    n, f2 = y2.shape
    w = fcw_pad.shape[1]

    def body(a_ref, y2_ref, dis_ref, b_ref, fw_ref, fb_ref, o_ref):
        a = a_ref[0] + a_ref[1]
        dis = dis_ref[...]
        h = jnp.maximum(dis * (a + y2_ref[...]) + b_ref[...], 0.0)
        lp = jnp.dot(h, fw_ref[...], **_DOT) + fb_ref[...]
        m = jnp.max(lp, axis=1, keepdims=True)
        ssum = jnp.sum(jnp.exp(lp - m), axis=1, keepdims=True)
        o_ref[...] = lp - m - jnp.log(ssum)

    return pl.pallas_call(
        body,
        grid=(n // r,),
        in_specs=[
            pl.BlockSpec((NC, r, f2), lambda i: (0, i, 0)),
            pl.BlockSpec((r, f2), lambda i: (i, 0)),
            pl.BlockSpec((r, 1), lambda i: (i, 0)),
            pl.BlockSpec((1, f2), lambda i: (0, 0)),
            pl.BlockSpec((f2, w), lambda i: (0, 0)),
            pl.BlockSpec((1, w), lambda i: (0, 0)),
        ],
        out_specs=pl.BlockSpec((r, w), lambda i: (i, 0)),
        out_shape=jax.ShapeDtypeStruct((n, w), jnp.float32),
    )(agg2, y2, dis, b2, fcw_pad, fcb_pad)


def kernel(x, edge_index, W1, b1, W2, b2, fc_W, fc_b):
    n, f = x.shape
    e = edge_index.shape[1]
    f2 = W2.shape[1]
    ncls = fc_W.shape[1]
    r = 2000

    src = edge_index[0].astype(jnp.int32)
    dst = edge_index[1].astype(jnp.int32)
    e_pad = -(-e // (NW * CH)) * (NW * CH)
    chunks = e_pad // CH
    pad = e_pad - e
    srcs = jnp.concatenate([src, jnp.zeros((pad,), jnp.int32)]).reshape(chunks, CH)
    dsts = jnp.concatenate([dst, jnp.full((pad,), n, jnp.int32)]).reshape(chunks, CH)

    n_pad = -(-(n + 1) // NS) * NS
    z16 = jnp.zeros((n_pad, 16), jnp.float32)
    zf = jnp.zeros((n_pad, f), jnp.float32)
    e0 = jnp.zeros((CH, 16), jnp.float32).at[:, 0].set(1.0)

    degout = _make_deg_kernel(n_pad, chunks)(dsts, e0, z16)
    y1, dis = _tc_pre(degout, x, W1, r)
    agg1 = _make_agg_kernel(n_pad, chunks, f)(srcs, dsts, y1, zf)
    y2 = _tc_mid(agg1[:, :n], y1, dis, b1.reshape(1, f), W2, r)
    agg2 = _make_agg_kernel(n_pad, chunks, f2)(srcs, dsts, y2, z16[:, :f2])
    fcw_pad = jnp.zeros((f2, 128), jnp.float32).at[:, :ncls].set(fc_W)
    fcb_pad = jnp.full((1, 128), -1e30, jnp.float32).at[0, :ncls].set(fc_b)
    out = _tc_post(agg2[:, :n], y2, dis, b2.reshape(1, f2), fcw_pad, fcb_pad, r)
    return out[:, :ncls]


# speedup vs baseline: 14.6743x; 14.6743x over previous
import jax
import jax.numpy as jnp
from jax import lax
from jax.experimental import pallas as pl
from jax.experimental.pallas import tpu as pltpu
from jax.experimental.pallas import tpu_sc as plsc

NC = 2
NS = 16
NW = NC * NS
CH = 128


def _mesh():
    return plsc.VectorSubcoreMesh(
        core_axis_name="c", subcore_axis_name="s", num_cores=NC, num_subcores=NS
    )


def _make_deg_kernel(n_pad, chunks):
    cpw = chunks // NW
    rpt = n_pad // NS

    def body(dsts, e0, zeros, out, idx_v, e0_v, acc):
        c = lax.axis_index("c")
        s = lax.axis_index("s")
        wid = s * NC + c
        pltpu.sync_copy(zeros.at[pl.ds(s * rpt, rpt)], acc.at[pl.ds(s * rpt, rpt)])
        pltpu.sync_copy(e0, e0_v)
        plsc.subcore_barrier()

        def step(j, carry):
            ch = wid * cpw + j
            pltpu.sync_copy(dsts.at[ch], idx_v)
            pltpu.sync_copy(e0_v, acc.at[idx_v], add=True)
            return carry

        lax.fori_loop(0, cpw, step, 0)
        plsc.subcore_barrier()
        pltpu.sync_copy(acc.at[pl.ds(s * rpt, rpt)], out.at[c, pl.ds(s * rpt, rpt)])

    return pl.kernel(
        body,
        out_type=jax.ShapeDtypeStruct((NC, n_pad, 16), jnp.float32),
        mesh=_mesh(),
        scratch_types=[
            pltpu.VMEM((CH,), jnp.int32),
            pltpu.VMEM((CH, 16), jnp.float32),
            pltpu.VMEM_SHARED((n_pad, 16), jnp.float32),
        ],
        compiler_params=pltpu.CompilerParams(use_tc_tiling_on_sc=False),
    )


def _make_agg_kernel(n_pad, chunks, d):
    cpw = chunks // NW
    rpt = n_pad // NS

    def body(srcs, dsts, y, zeros, out, sidx, didx, rows, acc):
        c = lax.axis_index("c")
        s = lax.axis_index("s")
        wid = s * NC + c
        pltpu.sync_copy(zeros.at[pl.ds(s * rpt, rpt)], acc.at[pl.ds(s * rpt, rpt)])
        plsc.subcore_barrier()

        def step(j, carry):
            ch = wid * cpw + j
            pltpu.sync_copy(srcs.at[ch], sidx)
            pltpu.sync_copy(dsts.at[ch], didx)
            pltpu.sync_copy(y.at[sidx], rows)
            pltpu.sync_copy(rows, acc.at[didx], add=True)
            return carry

        lax.fori_loop(0, cpw, step, 0)
        plsc.subcore_barrier()
        pltpu.sync_copy(acc.at[pl.ds(s * rpt, rpt)], out.at[c, pl.ds(s * rpt, rpt)])

    return pl.kernel(
        body,
        out_type=jax.ShapeDtypeStruct((NC, n_pad, d), jnp.float32),
        mesh=_mesh(),
        scratch_types=[
            pltpu.VMEM((CH,), jnp.int32),
            pltpu.VMEM((CH,), jnp.int32),
            pltpu.VMEM((CH, d), jnp.float32),
            pltpu.VMEM_SHARED((n_pad, d), jnp.float32),
        ],
        compiler_params=pltpu.CompilerParams(use_tc_tiling_on_sc=(d % 128 == 0)),
    )


_DOT = dict(preferred_element_type=jnp.float32, precision=lax.Precision.HIGHEST)


def _tc_pre(degout, x, w1, r):
    n, f = x.shape

    def body(deg_ref, x_ref, w_ref, y_ref, dis_ref):
        cnt = deg_ref[0, :, 0:1] + deg_ref[1, :, 0:1]
        dis = lax.rsqrt(cnt + 1.0)
        y_ref[...] = dis * jnp.dot(x_ref[...], w_ref[...], **_DOT)
        dis_ref[...] = dis

    return pl.pallas_call(
        body,
        grid=(n // r,),
        in_specs=[
            pl.BlockSpec((NC, r, 16), lambda i: (0, i, 0)),
            pl.BlockSpec((r, f), lambda i: (i, 0)),
            pl.BlockSpec((f, f), lambda i: (0, 0)),
        ],
        out_specs=[
            pl.BlockSpec((r, f), lambda i: (i, 0)),
            pl.BlockSpec((r, 1), lambda i: (i, 0)),
        ],
        out_shape=[
            jax.ShapeDtypeStruct((n, f), jnp.float32),
            jax.ShapeDtypeStruct((n, 1), jnp.float32),
        ],
    )(degout, x, w1)


def _tc_mid(agg1, y1, dis, b1, w2, r):
    n, f = y1.shape
    f2 = w2.shape[1]

    def body(a_ref, y1_ref, dis_ref, b_ref, w_ref, y2_ref):
        a = a_ref[0] + a_ref[1]
        dis = dis_ref[...]
        h = jnp.maximum(dis * (a + y1_ref[...]) + b_ref[...], 0.0)
        y2_ref[...] = dis * jnp.dot(h, w_ref[...], **_DOT)

    return pl.pallas_call(
        body,
        grid=(n // r,),
        in_specs=[
            pl.BlockSpec((NC, r, f), lambda i: (0, i, 0)),
            pl.BlockSpec((r, f), lambda i: (i, 0)),
            pl.BlockSpec((r, 1), lambda i: (i, 0)),
            pl.BlockSpec((1, f), lambda i: (0, 0)),
            pl.BlockSpec((f, f2), lambda i: (0, 0)),
        ],
        out_specs=pl.BlockSpec((r, f2), lambda i: (i, 0)),
        out_shape=jax.ShapeDtypeStruct((n, f2), jnp.float32),
    )(agg1, y1, dis, b1, w2)


def _tc_post(agg2, y2, dis, b2, fcw_pad, fcb_pad, r):
    n, f2 = y2.shape
    w = fcw_pad.shape[1]

    def body(a_ref, y2_ref, dis_ref, b_ref, fw_ref, fb_ref, o_ref):
        a = a_ref[0] + a_ref[1]
        dis = dis_ref[...]
        h = jnp.maximum(dis * (a + y2_ref[...]) + b_ref[...], 0.0)
        lp = jnp.dot(h, fw_ref[...], **_DOT) + fb_ref[...]
        m = jnp.max(lp, axis=1, keepdims=True)
        ssum = jnp.sum(jnp.exp(lp - m), axis=1, keepdims=True)
        o_ref[...] = lp - m - jnp.log(ssum)

    return pl.pallas_call(
        body,
        grid=(n // r,),
        in_specs=[
            pl.BlockSpec((NC, r, f2), lambda i: (0, i, 0)),
            pl.BlockSpec((r, f2), lambda i: (i, 0)),
            pl.BlockSpec((r, 1), lambda i: (i, 0)),
            pl.BlockSpec((1, f2), lambda i: (0, 0)),
            pl.BlockSpec((f2, w), lambda i: (0, 0)),
            pl.BlockSpec((1, w), lambda i: (0, 0)),
        ],
        out_specs=pl.BlockSpec((r, w), lambda i: (i, 0)),
        out_shape=jax.ShapeDtypeStruct((n, w), jnp.float32),
    )(agg2, y2, dis, b2, fcw_pad, fcb_pad)


def kernel(x, edge_index, W1, b1, W2, b2, fc_W, fc_b):
    n, f = x.shape
    e = edge_index.shape[1]
    f2 = W2.shape[1]
    ncls = fc_W.shape[1]
    r = 2000

    src = edge_index[0].astype(jnp.int32)
    dst = edge_index[1].astype(jnp.int32)
    e_pad = -(-e // (NW * CH)) * (NW * CH)
    chunks = e_pad // CH
    pad = e_pad - e
    srcs = jnp.concatenate([src, jnp.zeros((pad,), jnp.int32)]).reshape(chunks, CH)
    dsts = jnp.concatenate([dst, jnp.full((pad,), n, jnp.int32)]).reshape(chunks, CH)

    n_pad = -(-(n + 1) // (NS * 8)) * (NS * 8)
    z16 = jnp.zeros((n_pad, 16), jnp.float32)
    zf = jnp.zeros((n_pad, f), jnp.float32)
    e0 = jnp.zeros((CH, 16), jnp.float32).at[:, 0].set(1.0)

    degout = _make_deg_kernel(n_pad, chunks)(dsts, e0, z16)
    y1, dis = _tc_pre(degout, x, W1, r)
    agg1 = _make_agg_kernel(n_pad, chunks, f)(srcs, dsts, y1, zf)
    y2 = _tc_mid(agg1, y1, dis, b1.reshape(1, f), W2, r)
    agg2 = _make_agg_kernel(n_pad, chunks, f2)(srcs, dsts, y2, z16[:, :f2])
    fcw_pad = jnp.zeros((f2, 128), jnp.float32).at[:, :ncls].set(fc_W)
    fcb_pad = jnp.full((1, 128), -1e30, jnp.float32).at[0, :ncls].set(fc_b)
    out = _tc_post(agg2, y2, dis, b2.reshape(1, f2), fcw_pad, fcb_pad, r)
    return out[:, :ncls]
